# Initial kernel scaffold; baseline (speedup 1.0000x reference)
import jax
import jax.numpy as jnp
from jax.experimental import pallas as pl


def kernel(x, weight, weight_scale, weight_scale_2, bias):
    raise NotImplementedError("write your pallas kernel here")


# speedup vs baseline: 1956.5747x; 1956.5747x over previous
import functools

import jax
import jax.numpy as jnp
from jax.experimental import pallas as pl
from jax.experimental.pallas import tpu as pltpu

_BS = 16


def _fp4_rne(a):
    g = jnp.where(a > 0.25, 0.5, 0.0)
    g += jnp.where(a >= 0.75, 0.5, 0.0)
    g += jnp.where(a > 1.25, 0.5, 0.0)
    g += jnp.where(a >= 1.75, 0.5, 0.0)
    g += jnp.where(a > 2.5, 1.0, 0.0)
    g += jnp.where(a >= 3.5, 1.0, 0.0)
    g += jnp.where(a > 5.0, 2.0, 0.0)
    return g


def _qdq(t, scale):
    g = _fp4_rne(jnp.abs(t))
    return jnp.where(t < 0, -g, g) * scale


def _amax_kernel(x_ref, o_ref):
    m = jnp.max(jnp.abs(x_ref[...]))
    @pl.when(pl.program_id(0) == 0)
    def _init():
        o_ref[0, 0] = m
    @pl.when(pl.program_id(0) != 0)
    def _acc():
        o_ref[0, 0] = jnp.maximum(o_ref[0, 0], m)


def _wqdq_kernel(w_ref, sw_ref, o_ref):
    sw = sw_ref[...]
    o_ref[...] = _qdq(w_ref[...] / sw, sw)


def _main_kernel(x_ref, qw_ref, selb_ref, isc2_ref, bias_ref, o_ref):
    xt = x_ref[...]
    isc2 = isc2_ref[0, 0]
    m = jnp.abs(xt)
    k_lanes = xt.shape[1]
    for sh in (1, 2, 4, 8):
        m = jnp.maximum(m, pltpu.roll(m, k_lanes - sh, 1))
    pbs = m / 6.0
    q = pbs / isc2
    q = jnp.where(pbs == 0.0, 1.0, q)
    isc = jnp.clip(q, -448.0, 448.0)
    isc_rep = jax.lax.dot_general(
        isc, selb_ref[...], (((1,), (0,)), ((), ())),
        preferred_element_type=jnp.float32,
        precision=jax.lax.Precision.HIGHEST)
    s = isc_rep * isc2
    xdq = _qdq(xt / s, s)
    acc = jax.lax.dot_general(
        xdq, qw_ref[...], (((1,), (1,)), ((), ())),
        preferred_element_type=jnp.float32)
    o_ref[...] = acc + bias_ref[...]


@functools.partial(jax.jit, static_argnames=("interpret",))
def kernel(x, weight, weight_scale, weight_scale_2, bias, interpret=False):
    b, s, k = x.shape
    out_ch = weight.shape[0]
    m_total = b * s
    x2 = x.reshape(m_total, k)
    tile = 1024
    grid_m = m_total // tile

    amax = pl.pallas_call(
        _amax_kernel,
        grid=(grid_m,),
        in_specs=[pl.BlockSpec((tile, k), lambda i: (i, 0))],
        out_specs=pl.BlockSpec(memory_space=pltpu.SMEM),
        out_shape=jax.ShapeDtypeStruct((1, 1), jnp.float32),
        interpret=interpret,
    )(x2)

    sw_rep = jnp.repeat(weight_scale.astype(jnp.float32) * weight_scale_2,
                        _BS, axis=1)

    qdqw = pl.pallas_call(
        _wqdq_kernel,
        in_specs=[
            pl.BlockSpec((out_ch, k), lambda: (0, 0)),
            pl.BlockSpec((out_ch, k), lambda: (0, 0)),
        ],
        out_specs=pl.BlockSpec((out_ch, k), lambda: (0, 0)),
        out_shape=jax.ShapeDtypeStruct((out_ch, k), jnp.float32),
        interpret=interpret,
    )(weight, sw_rep)

    isc2 = amax[0, 0].astype(jnp.float32) / 6.0 / 448.0

    li = jnp.arange(k, dtype=jnp.int32)
    selb = (li[:, None] == _BS * (li[None, :] // _BS)).astype(jnp.float32)

    out = pl.pallas_call(
        _main_kernel,
        grid=(grid_m,),
        in_specs=[
            pl.BlockSpec((tile, k), lambda i: (i, 0)),
            pl.BlockSpec((out_ch, k), lambda i: (0, 0)),
            pl.BlockSpec((k, k), lambda i: (0, 0)),
            pl.BlockSpec(memory_space=pltpu.SMEM),
            pl.BlockSpec((1, out_ch), lambda i: (0, 0)),
        ],
        out_specs=pl.BlockSpec((tile, out_ch), lambda i: (i, 0)),
        out_shape=jax.ShapeDtypeStruct((m_total, out_ch), jnp.float32),
        compiler_params=pltpu.CompilerParams(
            dimension_semantics=("arbitrary",)),
        interpret=interpret,
    )(x2, qdqw, selb, isc2.reshape(1, 1), bias.reshape(1, out_ch))

    return out.reshape(b, s, out_ch)


# speedup vs baseline: 2696.9764x; 1.3784x over previous
import functools

import jax
import jax.numpy as jnp
from jax.experimental import pallas as pl
from jax.experimental.pallas import tpu as pltpu

_BS = 16


def _fp4_rne(a):
    g = jnp.where(a > 0.25, 0.5, 0.0)
    g += jnp.where(a >= 0.75, 0.5, 0.0)
    g += jnp.where(a > 1.25, 0.5, 0.0)
    g += jnp.where(a >= 1.75, 0.5, 0.0)
    g += jnp.where(a > 2.5, 1.0, 0.0)
    g += jnp.where(a >= 3.5, 1.0, 0.0)
    g += jnp.where(a > 5.0, 2.0, 0.0)
    return g


def _qdq(t, scale):
    g = _fp4_rne(jnp.abs(t))
    return jnp.where(t < 0, -g, g) * scale


def _amax_kernel(x_ref, o_ref):
    m = jnp.max(jnp.abs(x_ref[...]))
    @pl.when(pl.program_id(0) == 0)
    def _init():
        o_ref[0, 0] = m
    @pl.when(pl.program_id(0) != 0)
    def _acc():
        o_ref[0, 0] = jnp.maximum(o_ref[0, 0], m)


def _wqdq_kernel(w_ref, sw_ref, o_ref):
    sw = sw_ref[...]
    o_ref[...] = _qdq(w_ref[...] / sw, sw)


def _main_kernel(x_ref, qw_ref, isc2_ref, bias_ref, o_ref):
    xt = x_ref[...]
    isc2 = isc2_ref[0, 0]
    m = jnp.abs(xt)
    k_lanes = xt.shape[1]
    for sh in (1, 2, 4, 8):
        m = jnp.maximum(m, pltpu.roll(m, k_lanes - sh, 1))
    pbs = m / 6.0
    q = pbs / isc2
    q = jnp.where(pbs == 0.0, 1.0, q)
    isc = jnp.clip(q, -448.0, 448.0)
    sx = isc * isc2
    lane = jax.lax.broadcasted_iota(jnp.int32, xt.shape, 1)
    s = jnp.where(lane % _BS == 0, sx, 0.0)
    for sh in (1, 2, 4, 8):
        s = s + pltpu.roll(s, sh, 1)
    xdq = _qdq(xt / s, s)
    acc = jax.lax.dot_general(
        xdq, qw_ref[...], (((1,), (1,)), ((), ())),
        preferred_element_type=jnp.float32)
    o_ref[...] = acc + bias_ref[...]


@functools.partial(jax.jit, static_argnames=("interpret",))
def kernel(x, weight, weight_scale, weight_scale_2, bias, interpret=False):
    b, s, k = x.shape
    out_ch = weight.shape[0]
    m_total = b * s
    x2 = x.reshape(m_total, k)
    tile = 1024
    grid_m = m_total // tile

    amax = pl.pallas_call(
        _amax_kernel,
        grid=(grid_m,),
        in_specs=[pl.BlockSpec((tile, k), lambda i: (i, 0))],
        out_specs=pl.BlockSpec(memory_space=pltpu.SMEM),
        out_shape=jax.ShapeDtypeStruct((1, 1), jnp.float32),
        interpret=interpret,
    )(x2)

    sw_rep = jnp.repeat(weight_scale.astype(jnp.float32) * weight_scale_2,
                        _BS, axis=1)

    qdqw = pl.pallas_call(
        _wqdq_kernel,
        in_specs=[
            pl.BlockSpec((out_ch, k), lambda: (0, 0)),
            pl.BlockSpec((out_ch, k), lambda: (0, 0)),
        ],
        out_specs=pl.BlockSpec((out_ch, k), lambda: (0, 0)),
        out_shape=jax.ShapeDtypeStruct((out_ch, k), jnp.float32),
        interpret=interpret,
    )(weight, sw_rep)

    isc2 = amax[0, 0].astype(jnp.float32) / 6.0 / 448.0

    out = pl.pallas_call(
        _main_kernel,
        grid=(grid_m,),
        in_specs=[
            pl.BlockSpec((tile, k), lambda i: (i, 0)),
            pl.BlockSpec((out_ch, k), lambda i: (0, 0)),
            pl.BlockSpec(memory_space=pltpu.SMEM),
            pl.BlockSpec((1, out_ch), lambda i: (0, 0)),
        ],
        out_specs=pl.BlockSpec((tile, out_ch), lambda i: (i, 0)),
        out_shape=jax.ShapeDtypeStruct((m_total, out_ch), jnp.float32),
        compiler_params=pltpu.CompilerParams(
            dimension_semantics=("arbitrary",)),
        interpret=interpret,
    )(x2, qdqw, isc2.reshape(1, 1), bias.reshape(1, out_ch))

    return out.reshape(b, s, out_ch)


# speedup vs baseline: 2729.5310x; 1.0121x over previous
import functools

import jax
import jax.numpy as jnp
from jax.experimental import pallas as pl
from jax.experimental.pallas import tpu as pltpu

_BS = 16


def _fp4_rne(a):
    g = jnp.where(a > 0.25, 0.5, 0.0)
    g += jnp.where(a >= 0.75, 0.5, 0.0)
    g += jnp.where(a > 1.25, 0.5, 0.0)
    g += jnp.where(a >= 1.75, 0.5, 0.0)
    g += jnp.where(a > 2.5, 1.0, 0.0)
    g += jnp.where(a >= 3.5, 1.0, 0.0)
    g += jnp.where(a > 5.0, 2.0, 0.0)
    return g


def _qdq(t, scale):
    g = _fp4_rne(jnp.abs(t))
    return jnp.where(t < 0, -g, g) * scale


def _amax_kernel(x_ref, o_ref):
    m = jnp.max(jnp.abs(x_ref[...]))
    @pl.when(pl.program_id(0) == 0)
    def _init():
        o_ref[0, 0] = m
    @pl.when(pl.program_id(0) != 0)
    def _acc():
        o_ref[0, 0] = jnp.maximum(o_ref[0, 0], m)


def _wqdq_kernel(w_ref, sw_ref, o_ref):
    sw = sw_ref[...]
    o_ref[...] = _qdq(w_ref[...] / sw, sw)


def _main_kernel(x_ref, qw_ref, isc2_ref, bias_ref, o_ref):
    xt = x_ref[...]
    isc2 = isc2_ref[0, 0]
    m = jnp.abs(xt)
    k_lanes = xt.shape[1]
    for sh in (1, 2, 4, 8):
        m = jnp.maximum(m, pltpu.roll(m, k_lanes - sh, 1))
    pbs = m / 6.0
    q = pbs / isc2
    q = jnp.where(pbs == 0.0, 1.0, q)
    isc = jnp.clip(q, -448.0, 448.0)
    sx = isc * isc2
    lane = jax.lax.broadcasted_iota(jnp.int32, xt.shape, 1)
    s = jnp.where(lane % _BS == 0, sx, 0.0)
    for sh in (1, 2, 4, 8):
        s = s + pltpu.roll(s, sh, 1)
    xdq = _qdq(xt / s, s)
    acc = jax.lax.dot_general(
        xdq, qw_ref[...], (((1,), (1,)), ((), ())),
        preferred_element_type=jnp.float32)
    o_ref[...] = acc + bias_ref[...]


@functools.partial(jax.jit, static_argnames=("interpret",))
def kernel(x, weight, weight_scale, weight_scale_2, bias, interpret=False):
    b, s, k = x.shape
    out_ch = weight.shape[0]
    m_total = b * s
    x2 = x.reshape(m_total, k)
    tile = 2048
    grid_m = m_total // tile

    amax = pl.pallas_call(
        _amax_kernel,
        grid=(grid_m,),
        in_specs=[pl.BlockSpec((tile, k), lambda i: (i, 0))],
        out_specs=pl.BlockSpec(memory_space=pltpu.SMEM),
        out_shape=jax.ShapeDtypeStruct((1, 1), jnp.float32),
        interpret=interpret,
    )(x2)

    sw_rep = jnp.repeat(weight_scale.astype(jnp.float32) * weight_scale_2,
                        _BS, axis=1)

    qdqw = pl.pallas_call(
        _wqdq_kernel,
        in_specs=[
            pl.BlockSpec((out_ch, k), lambda: (0, 0)),
            pl.BlockSpec((out_ch, k), lambda: (0, 0)),
        ],
        out_specs=pl.BlockSpec((out_ch, k), lambda: (0, 0)),
        out_shape=jax.ShapeDtypeStruct((out_ch, k), jnp.float32),
        interpret=interpret,
    )(weight, sw_rep)

    isc2 = amax[0, 0].astype(jnp.float32) / 6.0 / 448.0

    out = pl.pallas_call(
        _main_kernel,
        grid=(grid_m,),
        in_specs=[
            pl.BlockSpec((tile, k), lambda i: (i, 0)),
            pl.BlockSpec((out_ch, k), lambda i: (0, 0)),
            pl.BlockSpec(memory_space=pltpu.SMEM),
            pl.BlockSpec((1, out_ch), lambda i: (0, 0)),
        ],
        out_specs=pl.BlockSpec((tile, out_ch), lambda i: (i, 0)),
        out_shape=jax.ShapeDtypeStruct((m_total, out_ch), jnp.float32),
        compiler_params=pltpu.CompilerParams(
            dimension_semantics=("arbitrary",)),
        interpret=interpret,
    )(x2, qdqw, isc2.reshape(1, 1), bias.reshape(1, out_ch))

    return out.reshape(b, s, out_ch)
